# Initial kernel scaffold; baseline (speedup 1.0000x reference)
import jax
import jax.numpy as jnp
from jax.experimental import pallas as pl


def kernel(q_pred, true_action, discounted_reward):
    raise NotImplementedError("write your pallas kernel here")


# speedup vs baseline: 1.9187x; 1.9187x over previous
import functools

import jax
import jax.numpy as jnp
from jax import lax
from jax.experimental import pallas as pl
from jax.experimental.pallas import tpu as pltpu
from jax.experimental.pallas import tpu_sc as plsc

B = 16384
NUM_ACTIONS = 6
NC, NS, L = 2, 16, 16
NW = NC * NS
ROWS_PER_TILE = B // NW
CHUNKS = ROWS_PER_TILE // L

_LN2 = 0.6931471805599453
_SQRT2 = 1.4142135623730951


def _log_f32(x):
    xi = plsc.bitcast(x, jnp.int32)
    e = (xi >> 23) - 127
    m = plsc.bitcast((xi & 0x007FFFFF) | 0x3F800000, jnp.float32)
    big = m > _SQRT2
    m = jnp.where(big, m * 0.5, m)
    e = e + jnp.where(big, 1, 0)
    s = (m - 1.0) / (m + 1.0)
    z = s * s
    ln_m = s * (2.0 + z * (2.0 / 3.0 + z * (0.4 + z * (2.0 / 7.0))))
    return e.astype(jnp.float32) * _LN2 + ln_m


def _sc_body(q_hbm, a_hbm, r_hbm, out_hbm, q_v, a_v, r_v, acc_v):
    wid = lax.axis_index("s") * NC + lax.axis_index("c")
    base = wid * ROWS_PER_TILE
    pltpu.sync_copy(q_hbm.at[pl.ds(base * NUM_ACTIONS, ROWS_PER_TILE * NUM_ACTIONS)], q_v)
    pltpu.sync_copy(a_hbm.at[pl.ds(base, ROWS_PER_TILE)], a_v)
    pltpu.sync_copy(r_hbm.at[pl.ds(base, ROWS_PER_TILE)], r_v)

    acc = jnp.zeros((L,), jnp.float32)
    for c in range(CHUNKS):
        rows = c * L + lax.iota(jnp.int32, L)
        acts = a_v[pl.ds(c * L, L)]
        rew = r_v[pl.ds(c * L, L)]
        g = plsc.load_gather(q_v, [rows * NUM_ACTIONS + acts])
        acc = acc + _log_f32(g) * rew

    acc_v[...] = acc
    pltpu.sync_copy(acc_v, out_hbm.at[wid])


_sc_partials = functools.partial(
    pl.kernel,
    mesh=plsc.VectorSubcoreMesh(core_axis_name="c", subcore_axis_name="s"),
    out_type=jax.ShapeDtypeStruct((NW, L), jnp.float32),
    compiler_params=pltpu.CompilerParams(needs_layout_passes=False),
    scratch_types=[
        pltpu.VMEM((ROWS_PER_TILE * NUM_ACTIONS,), jnp.float32),
        pltpu.VMEM((ROWS_PER_TILE,), jnp.int32),
        pltpu.VMEM((ROWS_PER_TILE,), jnp.float32),
        pltpu.VMEM((L,), jnp.float32),
    ],
)(_sc_body)


def _finish_body(p_ref, o_ref):
    o_ref[...] = (jnp.sum(p_ref[...]) * (-1.0 / B)).reshape(1, 1)


_finish = pl.pallas_call(
    _finish_body,
    out_shape=jax.ShapeDtypeStruct((1, 1), jnp.float32),
)


def kernel(q_pred, true_action, discounted_reward):
    partials = _sc_partials(
        q_pred.reshape(B * NUM_ACTIONS), true_action.reshape(B), discounted_reward
    )
    return _finish(partials)[0, 0]
